# baseline (speedup 1.0000x reference)
import functools

import jax
import jax.numpy as jnp
from jax import lax
from jax.experimental import pallas as pl
from jax.experimental.pallas import tpu as pltpu
from jax.experimental.pallas import tpu_sc as plsc

VOCAB = 1000
NC = 2
NS = 16
NW = NC * NS
LANES = 16
SEG = 2048


def _gather_clip_body(batch, fields, table_hbm, idxt_hbm, out_hbm,
                      table_v, idx_v, out_v, sem):
    wid = lax.axis_index("s") * NC + lax.axis_index("c")
    n_per_w = (batch * fields) // NW
    nseg = n_per_w // SEG
    segs_per_row = batch // SEG

    copies = []
    for j in range(nseg):
        m = nseg * wid + j
        f = m // segs_per_row
        b = SEG * lax.rem(m, segs_per_row)
        copies.append(pltpu.async_copy(
            idxt_hbm.at[pl.ds(f, 1), pl.ds(b, SEG)],
            idx_v.at[pl.ds(j, 1)], sem))

    pltpu.sync_copy(table_hbm, table_v)

    def clip_at(off):
        t = table_v[pl.ds(off, LANES)]
        table_v[pl.ds(off, LANES)] = jnp.minimum(jnp.maximum(t, -1.0), 1.0)

    @plsc.parallel_loop(0, VOCAB // LANES, unroll=4)
    def clip_body(j):
        clip_at(j * LANES)

    clip_at(VOCAB - LANES)

    for c in copies:
        c.wait()

    @plsc.parallel_loop(0, nseg, unroll=5)
    def gather_seg(j):
        for k in range(SEG // LANES):
            iv = idx_v[j, pl.ds(k * LANES, LANES)]
            out_v[pl.ds(j * SEG + k * LANES, LANES)] = plsc.load_gather(
                table_v, [iv])

    pltpu.sync_copy(out_v, out_hbm.at[pl.ds(wid * n_per_w, n_per_w)])


@functools.partial(jax.jit, static_argnames=("batch", "fields"))
def _run(idxt, table, batch, fields):
    n_per_w = (batch * fields) // NW
    mesh = plsc.VectorSubcoreMesh(core_axis_name="c", subcore_axis_name="s")
    body = functools.partial(_gather_clip_body, batch, fields)
    return pl.kernel(
        body,
        out_type=jax.ShapeDtypeStruct((batch * fields,), jnp.float32),
        mesh=mesh,
        scratch_types=[
            pltpu.VMEM((VOCAB,), jnp.float32),
            pltpu.VMEM((n_per_w // SEG, SEG), jnp.int32),
            pltpu.VMEM((n_per_w,), jnp.float32),
            pltpu.SemaphoreType.DMA,
        ],
        compiler_params=pltpu.CompilerParams(needs_layout_passes=False),
    )(table, idxt)


def kernel(idx, depth):
    b, f = idx.shape
    flat = _run(idx.T, depth.reshape((VOCAB,)), b, f)
    return jnp.transpose(flat.reshape((f, b, 1)), (1, 0, 2))


# speedup vs baseline: 1.3439x; 1.3439x over previous
import functools

import jax
import jax.numpy as jnp
from jax import lax
from jax.experimental import pallas as pl
from jax.experimental.pallas import tpu as pltpu
from jax.experimental.pallas import tpu_sc as plsc

VOCAB = 1000
NC = 2
NS = 16
NW = NC * NS
LANES = 16
SEG = 2048


def _gather_clip_body(batch, fields, table_hbm, idxt_hbm, out_hbm,
                      table_v, idx_v, out_v, sem):
    wid = lax.axis_index("s") * NC + lax.axis_index("c")
    n_per_w = (batch * fields) // NW
    nseg = n_per_w // SEG
    segs_per_row = batch // SEG

    copies = []
    for j in range(nseg):
        m = nseg * wid + j
        f = m // segs_per_row
        b = SEG * lax.rem(m, segs_per_row)
        copies.append(pltpu.async_copy(
            idxt_hbm.at[pl.ds(f, 1), pl.ds(b, SEG)],
            idx_v.at[pl.ds(j, 1)], sem))

    pltpu.sync_copy(table_hbm, table_v)

    def clip_at(off):
        t = table_v[pl.ds(off, LANES)]
        table_v[pl.ds(off, LANES)] = jnp.minimum(jnp.maximum(t, -1.0), 1.0)

    @plsc.parallel_loop(0, VOCAB // LANES, unroll=4)
    def clip_body(j):
        clip_at(j * LANES)

    clip_at(VOCAB - LANES)

    for c in copies:
        c.wait()

    @plsc.parallel_loop(0, nseg)
    def gather_seg(j):
        for k in range(SEG // LANES):
            iv = idx_v[j, pl.ds(k * LANES, LANES)]
            out_v[pl.ds(j * SEG + k * LANES, LANES)] = plsc.load_gather(
                table_v, [iv])

    pltpu.sync_copy(out_v, out_hbm.at[pl.ds(wid * n_per_w, n_per_w)])


@functools.partial(jax.jit, static_argnames=("batch", "fields"))
def _run(idxt, table, batch, fields):
    n_per_w = (batch * fields) // NW
    mesh = plsc.VectorSubcoreMesh(core_axis_name="c", subcore_axis_name="s")
    body = functools.partial(_gather_clip_body, batch, fields)
    return pl.kernel(
        body,
        out_type=jax.ShapeDtypeStruct((batch * fields,), jnp.float32),
        mesh=mesh,
        scratch_types=[
            pltpu.VMEM((VOCAB,), jnp.float32),
            pltpu.VMEM((n_per_w // SEG, SEG), jnp.int32),
            pltpu.VMEM((n_per_w,), jnp.float32),
            pltpu.SemaphoreType.DMA,
        ],
        compiler_params=pltpu.CompilerParams(needs_layout_passes=False),
    )(table, idxt)


def kernel(idx, depth):
    b, f = idx.shape
    flat = _run(idx.T, depth.reshape((VOCAB,)), b, f)
    return jnp.transpose(flat.reshape((f, b, 1)), (1, 0, 2))
